# baseline (speedup 1.0000x reference)
import functools

import jax
import jax.numpy as jnp
from jax import lax
from jax.experimental import pallas as pl
from jax.experimental.pallas import tpu as pltpu

R = 256
W = 264
PAD = 1280


def _fused_kernel(carr, ids_ref, prev_ref, idsc_ref, prevc_ref, triu_ref,
                  tril_ref, x_ref, emb_ref, wsrc_ref, bsrc_ref, wtgt_ref,
                  btgt_ref, xout_ref, iout_ref, p_sc, sums_sc, cnts_sc,
                  *, nb, H, S):
    i = pl.program_id(0)
    c = carr[i]
    base = (c // 8) * 8
    off = (c - base).astype(jnp.float32)

    @pl.when(i == 0)
    def _init():
        p_sc[...] = jnp.zeros_like(p_sc)
        sums_sc[...] = jnp.zeros_like(sums_sc)
        cnts_sc[...] = jnp.zeros_like(cnts_sc)
        p_sc[0:S, :] = jnp.dot(emb_ref[...].astype(jnp.bfloat16),
                               wsrc_ref[H:2 * H, :].astype(jnp.bfloat16),
                               preferred_element_type=jnp.float32
                               ).astype(jnp.bfloat16)

    bnd_r = (ids_ref[0] != prev_ref[0]).astype(jnp.bfloat16)
    bnd_c = (idsc_ref[0] != prevc_ref[0]).astype(jnp.bfloat16)
    seg_row = jnp.dot(bnd_r, triu_ref[...],
                      preferred_element_type=jnp.float32)
    seg_col = jnp.dot(tril_ref[...], bnd_c,
                      preferred_element_type=jnp.float32)

    rel_c = (seg_col + off).astype(jnp.int32)
    rel_r = (seg_row + off).astype(jnp.int32)
    onehot = (lax.broadcasted_iota(jnp.int32, (R, W), 1) == rel_c
              ).astype(jnp.bfloat16)
    onehot_t = (lax.broadcasted_iota(jnp.int32, (W, R), 0) == rel_r
                ).astype(jnp.bfloat16)

    xb = x_ref[...].astype(jnp.bfloat16)
    p_win = p_sc[pl.ds(base, W), :]
    gathered = jnp.dot(onehot, p_win, preferred_element_type=jnp.float32)
    xout_ref[...] = (jnp.dot(xb, wsrc_ref[0:H, :].astype(jnp.bfloat16),
                             preferred_element_type=jnp.float32)
                     + gathered + bsrc_ref[...])

    sums_sc[pl.ds(base, W), :] += jnp.dot(
        onehot_t, xb, preferred_element_type=jnp.float32)
    cnts_sc[pl.ds(base, W), 0:1] += jnp.sum(
        onehot_t, axis=1, keepdims=True).astype(jnp.float32)

    @pl.when(i == nb - 1)
    def _finish():
        n_runs = carr[nb] + 1
        means = sums_sc[0:S, :] / jnp.maximum(cnts_sc[0:S, 0:1], 1.0)
        sidx = lax.broadcasted_iota(jnp.int32, (S, 1), 0)
        emb = emb_ref[...]
        second = jnp.where(sidx < (n_runs - 1), means, emb)
        iout_ref[...] = (
            jnp.dot(emb.astype(jnp.bfloat16),
                    wtgt_ref[0:H, :].astype(jnp.bfloat16),
                    preferred_element_type=jnp.float32)
            + jnp.dot(second.astype(jnp.bfloat16),
                      wtgt_ref[H:2 * H, :].astype(jnp.bfloat16),
                      preferred_element_type=jnp.float32)
            + btgt_ref[...])


@jax.jit
def kernel(x_src, imputed_embs, src_ids, W_src, b_src, W_tgt, b_tgt):
    N, H = x_src.shape
    S = imputed_embs.shape[0]
    nb = N // R

    prev_ids = jnp.concatenate([src_ids[:1], src_ids[:-1]])
    bnd = (src_ids != prev_ids).astype(jnp.int32)
    cums = jnp.cumsum(bnd)
    carr = jnp.concatenate(
        [jnp.zeros((1,), jnp.int32), cums[R - 1::R].astype(jnp.int32)])

    ids3 = src_ids.reshape(nb, 1, R)
    prev3 = prev_ids.reshape(nb, 1, R)
    ids3c = src_ids.reshape(nb, R, 1)
    prev3c = prev_ids.reshape(nb, R, 1)

    io_r = lax.broadcasted_iota(jnp.int32, (R, R), 0)
    io_c = lax.broadcasted_iota(jnp.int32, (R, R), 1)
    tri_u = (io_r <= io_c).astype(jnp.bfloat16)
    tri_l = (io_c <= io_r).astype(jnp.bfloat16)

    grid_spec = pltpu.PrefetchScalarGridSpec(
        num_scalar_prefetch=1,
        grid=(nb,),
        in_specs=[
            pl.BlockSpec((1, 1, R), lambda i, c: (i, 0, 0)),
            pl.BlockSpec((1, 1, R), lambda i, c: (i, 0, 0)),
            pl.BlockSpec((1, R, 1), lambda i, c: (i, 0, 0)),
            pl.BlockSpec((1, R, 1), lambda i, c: (i, 0, 0)),
            pl.BlockSpec((R, R), lambda i, c: (0, 0)),
            pl.BlockSpec((R, R), lambda i, c: (0, 0)),
            pl.BlockSpec((R, H), lambda i, c: (i, 0)),
            pl.BlockSpec((S, H), lambda i, c: (0, 0)),
            pl.BlockSpec((2 * H, H), lambda i, c: (0, 0)),
            pl.BlockSpec((1, H), lambda i, c: (0, 0)),
            pl.BlockSpec((2 * H, H), lambda i, c: (0, 0)),
            pl.BlockSpec((1, H), lambda i, c: (0, 0)),
        ],
        out_specs=[
            pl.BlockSpec((R, H), lambda i, c: (i, 0)),
            pl.BlockSpec((S, H), lambda i, c: (0, 0)),
        ],
        scratch_shapes=[
            pltpu.VMEM((PAD, H), jnp.bfloat16),
            pltpu.VMEM((PAD, H), jnp.float32),
            pltpu.VMEM((PAD, 128), jnp.float32),
        ],
    )

    x_out, imputed_out = pl.pallas_call(
        functools.partial(_fused_kernel, nb=nb, H=H, S=S),
        grid_spec=grid_spec,
        out_shape=[
            jax.ShapeDtypeStruct((N, H), jnp.float32),
            jax.ShapeDtypeStruct((S, H), jnp.float32),
        ],
        compiler_params=pltpu.CompilerParams(
            dimension_semantics=("arbitrary",)),
    )(carr, ids3, prev3, ids3c, prev3c, tri_u, tri_l, x_src, imputed_embs,
      W_src, b_src.reshape(1, H), W_tgt, b_tgt.reshape(1, H))
    return (x_out, imputed_out)


# speedup vs baseline: 1.0000x; 1.0000x over previous
import functools

import jax
import jax.numpy as jnp
from jax import lax
from jax.experimental import pallas as pl
from jax.experimental.pallas import tpu as pltpu

R = 256
W = 272
PAD = 1280


def _fused_kernel(carr, ids_ref, prev_ref, idsc_ref, prevc_ref, triu_ref,
                  tril_ref, x_ref, emb_ref, wsrc_ref, bsrc_ref, wtgt_ref,
                  btgt_ref, xout_ref, iout_ref, p_sc, sums_sc, cnts_sc,
                  *, nb, H, S):
    i = pl.program_id(0)
    c = carr[i]
    base = (c // 16) * 16
    off = (c - base).astype(jnp.float32)

    @pl.when(i == 0)
    def _init():
        p_sc[...] = jnp.zeros_like(p_sc)
        sums_sc[...] = jnp.zeros_like(sums_sc)
        cnts_sc[...] = jnp.zeros_like(cnts_sc)
        p_sc[0:S, :] = jnp.dot(emb_ref[...].astype(jnp.bfloat16),
                               wsrc_ref[H:2 * H, :].astype(jnp.bfloat16),
                               preferred_element_type=jnp.float32
                               ).astype(jnp.bfloat16)

    bnd_r = (ids_ref[0] != prev_ref[0]).astype(jnp.bfloat16)
    bnd_c = (idsc_ref[0] != prevc_ref[0]).astype(jnp.bfloat16)
    seg_row = jnp.dot(bnd_r, triu_ref[...],
                      preferred_element_type=jnp.float32)
    seg_col = jnp.dot(tril_ref[...], bnd_c,
                      preferred_element_type=jnp.float32)

    rel_c = (seg_col + off).astype(jnp.int32)
    rel_r = (seg_row + off).astype(jnp.int32)
    onehot = (lax.broadcasted_iota(jnp.int32, (R, W), 1) == rel_c
              ).astype(jnp.bfloat16)
    onehot_t = (lax.broadcasted_iota(jnp.int32, (W, R), 0) == rel_r
                ).astype(jnp.bfloat16)

    xb = x_ref[...].astype(jnp.bfloat16)
    p_win = p_sc[pl.ds(base, W), :]
    gathered = jnp.dot(onehot, p_win, preferred_element_type=jnp.float32)
    xout_ref[...] = (jnp.dot(xb, wsrc_ref[0:H, :].astype(jnp.bfloat16),
                             preferred_element_type=jnp.float32)
                     + gathered + bsrc_ref[...])

    sums_sc[pl.ds(base, W), :] += jnp.dot(
        onehot_t, xb, preferred_element_type=jnp.float32)
    cnts_sc[pl.ds(base, W), 0:1] += jnp.sum(
        onehot_t, axis=1, keepdims=True).astype(jnp.float32)

    @pl.when(i == nb - 1)
    def _finish():
        n_runs = carr[nb] + 1
        means = sums_sc[0:S, :] / jnp.maximum(cnts_sc[0:S, 0:1], 1.0)
        sidx = lax.broadcasted_iota(jnp.int32, (S, 1), 0)
        emb = emb_ref[...]
        second = jnp.where(sidx < (n_runs - 1), means, emb)
        iout_ref[...] = (
            jnp.dot(emb.astype(jnp.bfloat16),
                    wtgt_ref[0:H, :].astype(jnp.bfloat16),
                    preferred_element_type=jnp.float32)
            + jnp.dot(second.astype(jnp.bfloat16),
                      wtgt_ref[H:2 * H, :].astype(jnp.bfloat16),
                      preferred_element_type=jnp.float32)
            + btgt_ref[...])


@jax.jit
def kernel(x_src, imputed_embs, src_ids, W_src, b_src, W_tgt, b_tgt):
    N, H = x_src.shape
    S = imputed_embs.shape[0]
    nb = N // R

    prev_ids = jnp.concatenate([src_ids[:1], src_ids[:-1]])
    bnd = (src_ids != prev_ids).astype(jnp.int32)
    cums = jnp.cumsum(bnd)
    carr = jnp.concatenate(
        [jnp.zeros((1,), jnp.int32), cums[R - 1::R].astype(jnp.int32)])

    ids3 = src_ids.reshape(nb, 1, R)
    prev3 = prev_ids.reshape(nb, 1, R)
    ids3c = src_ids.reshape(nb, R, 1)
    prev3c = prev_ids.reshape(nb, R, 1)

    io_r = lax.broadcasted_iota(jnp.int32, (R, R), 0)
    io_c = lax.broadcasted_iota(jnp.int32, (R, R), 1)
    tri_u = (io_r <= io_c).astype(jnp.bfloat16)
    tri_l = (io_c <= io_r).astype(jnp.bfloat16)

    grid_spec = pltpu.PrefetchScalarGridSpec(
        num_scalar_prefetch=1,
        grid=(nb,),
        in_specs=[
            pl.BlockSpec((1, 1, R), lambda i, c: (i, 0, 0)),
            pl.BlockSpec((1, 1, R), lambda i, c: (i, 0, 0)),
            pl.BlockSpec((1, R, 1), lambda i, c: (i, 0, 0)),
            pl.BlockSpec((1, R, 1), lambda i, c: (i, 0, 0)),
            pl.BlockSpec((R, R), lambda i, c: (0, 0)),
            pl.BlockSpec((R, R), lambda i, c: (0, 0)),
            pl.BlockSpec((R, H), lambda i, c: (i, 0)),
            pl.BlockSpec((S, H), lambda i, c: (0, 0)),
            pl.BlockSpec((2 * H, H), lambda i, c: (0, 0)),
            pl.BlockSpec((1, H), lambda i, c: (0, 0)),
            pl.BlockSpec((2 * H, H), lambda i, c: (0, 0)),
            pl.BlockSpec((1, H), lambda i, c: (0, 0)),
        ],
        out_specs=[
            pl.BlockSpec((R, H), lambda i, c: (i, 0)),
            pl.BlockSpec((S, H), lambda i, c: (0, 0)),
        ],
        scratch_shapes=[
            pltpu.VMEM((PAD, H), jnp.bfloat16),
            pltpu.VMEM((PAD, H), jnp.float32),
            pltpu.VMEM((PAD, 128), jnp.float32),
        ],
    )

    x_out, imputed_out = pl.pallas_call(
        functools.partial(_fused_kernel, nb=nb, H=H, S=S),
        grid_spec=grid_spec,
        out_shape=[
            jax.ShapeDtypeStruct((N, H), jnp.float32),
            jax.ShapeDtypeStruct((S, H), jnp.float32),
        ],
        compiler_params=pltpu.CompilerParams(
            dimension_semantics=("arbitrary",)),
    )(carr, ids3, prev3, ids3c, prev3c, tri_u, tri_l, x_src, imputed_embs,
      W_src, b_src.reshape(1, H), W_tgt, b_tgt.reshape(1, H))
    return (x_out, imputed_out)


# speedup vs baseline: 1.4472x; 1.4472x over previous
import functools

import jax
import jax.numpy as jnp
import numpy as np
from jax import lax
from jax.experimental import pallas as pl
from jax.experimental.pallas import tpu as pltpu

R = 256
W = 384
CW = 128
PAD = 1408

def _fused_kernel(carr, ids_ref, prev_ref, x_ref, emb_ref, wsrc_ref, bsrc_ref,
                  wtgt_ref, btgt_ref, xout_ref, iout_ref, p_sc, sums_sc,
                  *, nb, H, S):
    i = pl.program_id(0)
    c = carr[i]
    base = (c // 8) * 8
    off = (c - base).astype(jnp.float32)

    @pl.when(i == 0)
    def _init():
        p_sc[...] = jnp.zeros_like(p_sc)
        sums_sc[...] = jnp.zeros_like(sums_sc)
        p_sc[0:S, :] = jnp.dot(emb_ref[...].astype(jnp.bfloat16),
                               wsrc_ref[H:2 * H, :].astype(jnp.bfloat16),
                               preferred_element_type=jnp.float32)

    bnd = (ids_ref[0] != prev_ref[0]).astype(jnp.float32)
    io_r = lax.broadcasted_iota(jnp.int32, (R, R), 0)
    io_c = lax.broadcasted_iota(jnp.int32, (R, R), 1)
    tri_u = (io_r <= io_c).astype(jnp.float32)
    tri_l = (io_c <= io_r).astype(jnp.float32)
    seg_row = jnp.dot(bnd, tri_u, preferred_element_type=jnp.float32)
    seg_col = jnp.sum(tri_l * bnd, axis=1, keepdims=True)

    rel_c = (seg_col + off).astype(jnp.int32)
    rel_r = (seg_row + off).astype(jnp.int32)
    onehot = (lax.broadcasted_iota(jnp.int32, (R, W), 1) == rel_c
              ).astype(jnp.bfloat16)
    onehot_t = (lax.broadcasted_iota(jnp.int32, (W, R), 0) == rel_r
                ).astype(jnp.bfloat16)

    xb = x_ref[...].astype(jnp.bfloat16)
    p_win = p_sc[pl.ds(base, W), :].astype(jnp.bfloat16)
    gathered = jnp.dot(onehot, p_win, preferred_element_type=jnp.float32)
    xout_ref[...] = (jnp.dot(xb, wsrc_ref[0:H, :].astype(jnp.bfloat16),
                             preferred_element_type=jnp.float32)
                     + gathered + bsrc_ref[...])

    x_aug = jnp.concatenate(
        [xb, jnp.ones((R, CW), dtype=jnp.bfloat16)], axis=1)
    sums_sc[pl.ds(base, W), :] += jnp.dot(
        onehot_t, x_aug, preferred_element_type=jnp.float32)

    @pl.when(i == nb - 1)
    def _finish():
        n_runs = carr[nb] + 1
        sums = sums_sc[0:S, 0:H]
        cnt = sums_sc[0:S, H:H + 1]
        means = sums / jnp.maximum(cnt, 1.0)
        sidx = lax.broadcasted_iota(jnp.int32, (S, 1), 0)
        emb = emb_ref[...]
        second = jnp.where(sidx < (n_runs - 1), means, emb)
        iout_ref[...] = (
            jnp.dot(emb.astype(jnp.bfloat16),
                    wtgt_ref[0:H, :].astype(jnp.bfloat16),
                    preferred_element_type=jnp.float32)
            + jnp.dot(second.astype(jnp.bfloat16),
                      wtgt_ref[H:2 * H, :].astype(jnp.bfloat16),
                      preferred_element_type=jnp.float32)
            + btgt_ref[...])


@jax.jit
def kernel(x_src, imputed_embs, src_ids, W_src, b_src, W_tgt, b_tgt):
    N, H = x_src.shape
    S = imputed_embs.shape[0]
    nb = N // R

    prev_ids = jnp.concatenate([src_ids[:1], src_ids[:-1]])
    bnd2 = (src_ids.reshape(nb, R) != prev_ids.reshape(nb, R))
    blk_counts = jnp.sum(bnd2.astype(jnp.int32), axis=1)
    carr = jnp.concatenate(
        [jnp.zeros((1,), jnp.int32), jnp.cumsum(blk_counts)]).astype(jnp.int32)

    ids3 = src_ids.reshape(nb, 1, R)
    prev3 = prev_ids.reshape(nb, 1, R)

    grid_spec = pltpu.PrefetchScalarGridSpec(
        num_scalar_prefetch=1,
        grid=(nb,),
        in_specs=[
            pl.BlockSpec((1, 1, R), lambda i, c: (i, 0, 0)),
            pl.BlockSpec((1, 1, R), lambda i, c: (i, 0, 0)),
            pl.BlockSpec((R, H), lambda i, c: (i, 0)),
            pl.BlockSpec((S, H), lambda i, c: (0, 0)),
            pl.BlockSpec((2 * H, H), lambda i, c: (0, 0)),
            pl.BlockSpec((1, H), lambda i, c: (0, 0)),
            pl.BlockSpec((2 * H, H), lambda i, c: (0, 0)),
            pl.BlockSpec((1, H), lambda i, c: (0, 0)),
        ],
        out_specs=[
            pl.BlockSpec((R, H), lambda i, c: (i, 0)),
            pl.BlockSpec((S, H), lambda i, c: (0, 0)),
        ],
        scratch_shapes=[
            pltpu.VMEM((PAD, H), jnp.float32),
            pltpu.VMEM((PAD, H + CW), jnp.float32),
        ],
    )

    x_out, imputed_out = pl.pallas_call(
        functools.partial(_fused_kernel, nb=nb, H=H, S=S),
        grid_spec=grid_spec,
        out_shape=[
            jax.ShapeDtypeStruct((N, H), jnp.float32),
            jax.ShapeDtypeStruct((S, H), jnp.float32),
        ],
        compiler_params=pltpu.CompilerParams(
            dimension_semantics=("arbitrary",)),
    )(carr, ids3, prev3, x_src, imputed_embs, W_src,
      b_src.reshape(1, H), W_tgt, b_tgt.reshape(1, H))
    return (x_out, imputed_out)
